# baseline (speedup 1.0000x reference)
import functools

import jax
import jax.numpy as jnp
from jax import lax
from jax.experimental import pallas as pl
from jax.experimental.pallas import tpu as pltpu
from jax.experimental.pallas import tpu_sc as plsc

N = 100000
E = 1600000
NH = N // 2
EH = E // 2
IN_CH = 6
HID = 64
W = 2 * HID
MP = 2

FCH = 16
NCH = HID // FCH
EB = 100
NT = E // EB
NTH = EH // EB
NC = 2
NS = 16
NW = NC * NS
NBUF = 4
ZR = 250

NB2 = 2000
EB2 = 4000

_F32 = jnp.float32


def _full(shape):
    return pl.BlockSpec(shape, lambda i: tuple(0 for _ in shape))


def _bd(w):
    fi, fo = w.shape
    z = jnp.zeros((2 * fi, 2 * fo), w.dtype)
    return z.at[:fi, :fo].set(w).at[fi:, fo:].set(w)


def _bt(b):
    return jnp.concatenate([b, b]).reshape(1, W)


def _enc_body(x_ref, w1, b1, w2, b2, w3, b3, w1h, h_ref, hp_ref):
    z = jnp.maximum(jnp.dot(x_ref[...], w1[...], preferred_element_type=_F32) + b1[...], 0.0)
    z = jnp.maximum(jnp.dot(z, w2[...], preferred_element_type=_F32) + b2[...], 0.0)
    h = jnp.dot(z, w3[...], preferred_element_type=_F32) + b3[...]
    h_ref[...] = h
    hp_ref[...] = jnp.dot(h, w1h[...], preferred_element_type=_F32)


def _tc_encoder(x2, w1, b1, w2, b2, w3, b3, w1h):
    return pl.pallas_call(
        _enc_body,
        grid=(NH // NB2,),
        in_specs=[
            pl.BlockSpec((NB2, 2 * IN_CH), lambda i: (i, 0)),
            _full((2 * IN_CH, W)), _full((1, W)),
            _full((W, W)), _full((1, W)),
            _full((W, W)), _full((1, W)),
            _full((W, W)),
        ],
        out_specs=[pl.BlockSpec((NB2, W), lambda i: (i, 0))] * 2,
        out_shape=[jax.ShapeDtypeStruct((NH, W), _F32)] * 2,
    )(x2, w1, b1, w2, b2, w3, b3, w1h)


def _edge_body(g_ref, ealo_ref, eahi_ref, w1lo, w1hi, b1, w2, b2, w3, b3,
               m_ref):
    z = (g_ref[...] + b1[...]
         + jnp.dot(ealo_ref[...], w1lo[...], preferred_element_type=_F32)
         + jnp.dot(eahi_ref[...], w1hi[...], preferred_element_type=_F32))
    z = jnp.maximum(z, 0.0)
    z = jnp.maximum(jnp.dot(z, w2[...], preferred_element_type=_F32) + b2[...], 0.0)
    m_ref[...] = jnp.dot(z, w3[...], preferred_element_type=_F32) + b3[...]


def _tc_edge(g2, ea, w1lo, w1hi, b1, w2, b2, w3, b3):
    return pl.pallas_call(
        _edge_body,
        grid=(EH // EB2,),
        in_specs=[
            pl.BlockSpec((EB2, W), lambda i: (i, 0)),
            pl.BlockSpec((EB2, 4), lambda i: (i, 0)),
            pl.BlockSpec((EB2, 4), lambda i: (i + EH // EB2, 0)),
            _full((4, W)), _full((4, W)), _full((1, W)),
            _full((W, W)), _full((1, W)),
            _full((W, W)), _full((1, W)),
        ],
        out_specs=pl.BlockSpec((EB2, W), lambda i: (i, 0)),
        out_shape=jax.ShapeDtypeStruct((EH, W), _F32),
    )(g2, ea, ea, w1lo, w1hi, b1, w2, b2, w3, b3)


def _upd_body(h_ref, alo_ref, ahi_ref, u1h, u1lo, u1hi, b1, w2, b2, w3, b3,
              w1h, ho_ref, hp_ref):
    h = h_ref[...]
    acc = jnp.dot(h, u1h[...], preferred_element_type=_F32) + b1[...]
    for f in range(NCH):
        acc = acc + jnp.dot(alo_ref[f], u1lo[f], preferred_element_type=_F32)
        acc = acc + jnp.dot(ahi_ref[f], u1hi[f], preferred_element_type=_F32)
    z = jnp.maximum(acc, 0.0)
    z = jnp.maximum(jnp.dot(z, w2[...], preferred_element_type=_F32) + b2[...], 0.0)
    ho = h + jnp.dot(z, w3[...], preferred_element_type=_F32) + b3[...]
    ho_ref[...] = ho
    hp_ref[...] = jnp.dot(ho, w1h[...], preferred_element_type=_F32)


def _tc_update(h2, agg, u1h, u1lo, u1hi, b1, w2, b2, w3, b3, w1h):
    return pl.pallas_call(
        _upd_body,
        grid=(NH // NB2,),
        in_specs=[
            pl.BlockSpec((NB2, W), lambda i: (i, 0)),
            pl.BlockSpec((NCH, NB2, FCH), lambda i: (0, i, 0)),
            pl.BlockSpec((NCH, NB2, FCH), lambda i: (0, i + NH // NB2, 0)),
            _full((W, W)), _full((NCH, FCH, W)), _full((NCH, FCH, W)),
            _full((1, W)),
            _full((W, W)), _full((1, W)),
            _full((W, W)), _full((1, W)),
            _full((W, W)),
        ],
        out_specs=[pl.BlockSpec((NB2, W), lambda i: (i, 0))] * 2,
        out_shape=[jax.ShapeDtypeStruct((NH, W), _F32)] * 2,
    )(h2, agg, agg, u1h, u1lo, u1hi, b1, w2, b2, w3, b3, w1h)


def _sc_gather(tab, src2d):
    mesh = plsc.VectorSubcoreMesh(core_axis_name="c", subcore_axis_name="s")
    per_w = NT // NW
    grp = per_w // NBUF

    @functools.partial(
        pl.kernel,
        mesh=mesh,
        out_type=jax.ShapeDtypeStruct((EH, W), _F32),
        scratch_types=[pltpu.VMEM((NBUF, EB), jnp.int32),
                       pltpu.VMEM((NBUF * EB, HID), _F32)]
        + [pltpu.SemaphoreType.DMA] * 3,
        compiler_params=pltpu.CompilerParams(use_tc_tiling_on_sc=False),
    )
    def k(tab_hbm, src_hbm, g_hbm, idx_v, rows_v, isem, gsem, wsem):
        wid = lax.axis_index("s") * NC + lax.axis_index("c")
        h = wid // (NW // 2)
        base = wid * per_w

        def body(o, carry):
            t0 = base + o * NBUF
            jh0 = t0 - h * NTH
            pltpu.async_copy(src_hbm.at[pl.ds(t0, NBUF), :], idx_v,
                             isem).wait()
            ds = [pltpu.async_copy(tab_hbm.at[idx_v.at[b]],
                                   rows_v.at[pl.ds(b * EB, EB), :], gsem)
                  for b in range(NBUF)]
            for d in ds:
                d.wait()
            pltpu.async_copy(
                rows_v,
                g_hbm.at[pl.ds(jh0 * EB, NBUF * EB), pl.ds(h * HID, HID)],
                wsem).wait()
            return carry

        lax.fori_loop(0, grp, body, 0)

    return k(tab, src2d)


def _sc_scatter(m2, dst2d):
    mesh = plsc.VectorSubcoreMesh(core_axis_name="c", subcore_axis_name="s")
    per_t = NT // NS
    SB = 8
    grp = per_t // SB

    @functools.partial(
        pl.kernel,
        mesh=mesh,
        out_type=jax.ShapeDtypeStruct((NCH, N, FCH), _F32),
        scratch_types=[pltpu.VMEM((SB, EB), jnp.int32),
                       pltpu.VMEM((SB * EB, FCH), _F32),
                       pltpu.VMEM((ZR, FCH), _F32), pltpu.VMEM((ZR, FCH), _F32),
                       pltpu.VMEM_SHARED((N, FCH), _F32)]
        + [pltpu.SemaphoreType.DMA] * 2,
        compiler_params=pltpu.CompilerParams(use_tc_tiling_on_sc=False),
    )
    def k(m_hbm, dst_hbm, agg_hbm, idx_v, rows_v, zbuf_v, bounce_v, acc_sh,
          lsem, asem):
        c = lax.axis_index("c")
        s = lax.axis_index("s")
        row0 = s * (N // NS)
        h = s // (NS // 2)

        def zfill(i, carry):
            zbuf_v[i, :] = jnp.zeros((FCH,), _F32)
            return carry

        lax.fori_loop(0, ZR, zfill, 0)

        for q in range(NCH // NC):
            ch = c * (NCH // NC) + q

            def zcopy(i, carry):
                pltpu.sync_copy(zbuf_v, acc_sh.at[pl.ds(row0 + i * ZR, ZR), :])
                return carry

            lax.fori_loop(0, (N // NS) // ZR, zcopy, 0)
            plsc.subcore_barrier()

            def body(o, carry):
                t0 = s * per_t + o * SB
                jh0 = t0 - h * NTH
                d1 = pltpu.async_copy(dst_hbm.at[pl.ds(t0, SB), :], idx_v,
                                      lsem)
                d2 = pltpu.async_copy(
                    m_hbm.at[pl.ds(jh0 * EB, SB * EB),
                             pl.ds(h * HID + ch * FCH, FCH)],
                    rows_v, lsem)
                d1.wait()
                d2.wait()
                ds = [pltpu.async_copy(rows_v.at[pl.ds(b * EB, EB), :],
                                       acc_sh.at[idx_v.at[b]], asem,
                                       add=True)
                      for b in range(SB)]
                for d in ds:
                    d.wait()
                return carry

            lax.fori_loop(0, grp, body, 0)
            plsc.subcore_barrier()

            def flush(i, carry):
                r0_ = row0 + i * ZR
                pltpu.sync_copy(acc_sh.at[pl.ds(r0_, ZR), :], bounce_v)
                pltpu.sync_copy(bounce_v, agg_hbm.at[ch, pl.ds(r0_, ZR), :])
                return carry

            lax.fori_loop(0, (N // NS) // ZR, flush, 0)
            plsc.subcore_barrier()

    return k(m2, dst2d)


def kernel(x, ei, ea,
           enc_W1, enc_b1, enc_W2, enc_b2, enc_W3, enc_b3,
           edge_W1, edge_b1, edge_W2, edge_b2, edge_W3, edge_b3,
           upd_W1, upd_b1, upd_W2, upd_b2, upd_W3, upd_b3):
    src = ei[0]
    dst = ei[1]
    src_r = jnp.where(src < NH, 2 * src, 2 * (src - NH) + 1).astype(jnp.int32)
    src2d = src_r.reshape(NT, EB)
    dst2d = dst.reshape(NT, EB)

    x2 = jnp.concatenate([x[:NH], x[NH:]], axis=1)

    ew1 = _bd(enc_W1)
    ew2 = _bd(enc_W2)
    ew3 = _bd(enc_W3)
    eb1, eb2, eb3 = _bt(enc_b1), _bt(enc_b2), _bt(enc_b3)

    w1h = _bd(edge_W1[:HID])
    w1e = edge_W1[HID:]
    w1lo = jnp.zeros((4, W), _F32).at[:, :HID].set(w1e)
    w1hi = jnp.zeros((4, W), _F32).at[:, HID:].set(w1e)
    gw2 = _bd(edge_W2)
    gw3 = _bd(edge_W3)
    gb1, gb2, gb3 = _bt(edge_b1), _bt(edge_b2), _bt(edge_b3)

    u1h = _bd(upd_W1[:HID])
    u1a = upd_W1[HID:].reshape(NCH, FCH, HID)
    u1lo = jnp.zeros((NCH, FCH, W), _F32).at[:, :, :HID].set(u1a)
    u1hi = jnp.zeros((NCH, FCH, W), _F32).at[:, :, HID:].set(u1a)
    uw2 = _bd(upd_W2)
    uw3 = _bd(upd_W3)
    ub1, ub2, ub3 = _bt(upd_b1), _bt(upd_b2), _bt(upd_b3)

    h2, hp2 = _tc_encoder(x2, ew1, eb1, ew2, eb2, ew3, eb3, w1h)
    for _ in range(MP):
        g2 = _sc_gather(hp2.reshape(N, HID), src2d)
        m2 = _tc_edge(g2, ea, w1lo, w1hi, gb1, gw2, gb2, gw3, gb3)
        agg = _sc_scatter(m2, dst2d)
        h2, hp2 = _tc_update(h2, agg, u1h, u1lo, u1hi, ub1, uw2, ub2, uw3,
                             ub3, w1h)
    return jnp.concatenate([h2[:, :HID], h2[:, HID:]], axis=0)


# speedup vs baseline: 1.0132x; 1.0132x over previous
import functools

import jax
import jax.numpy as jnp
from jax import lax
from jax.experimental import pallas as pl
from jax.experimental.pallas import tpu as pltpu
from jax.experimental.pallas import tpu_sc as plsc

N = 100000
E = 1600000
NH = N // 2
EH = E // 2
IN_CH = 6
HID = 64
W = 2 * HID
MP = 2

FCH = 16
NCH = HID // FCH
EB = 100
NT = E // EB
NTH = EH // EB
NC = 2
NS = 16
NW = NC * NS
NBUF = 5
ZR = 250

NB2 = 2000
EB2 = 4000

_F32 = jnp.float32


def _full(shape):
    return pl.BlockSpec(shape, lambda i: tuple(0 for _ in shape))


def _bd(w):
    fi, fo = w.shape
    z = jnp.zeros((2 * fi, 2 * fo), w.dtype)
    return z.at[:fi, :fo].set(w).at[fi:, fo:].set(w)


def _bt(b):
    return jnp.concatenate([b, b]).reshape(1, W)


def _enc_body(x_ref, w1, b1, w2, b2, w3, b3, w1h, h_ref, hp_ref):
    z = jnp.maximum(jnp.dot(x_ref[...], w1[...], preferred_element_type=_F32) + b1[...], 0.0)
    z = jnp.maximum(jnp.dot(z, w2[...], preferred_element_type=_F32) + b2[...], 0.0)
    h = jnp.dot(z, w3[...], preferred_element_type=_F32) + b3[...]
    h_ref[...] = h
    hp_ref[...] = jnp.dot(h, w1h[...], preferred_element_type=_F32)


def _tc_encoder(x2, w1, b1, w2, b2, w3, b3, w1h):
    return pl.pallas_call(
        _enc_body,
        grid=(NH // NB2,),
        in_specs=[
            pl.BlockSpec((NB2, 2 * IN_CH), lambda i: (i, 0)),
            _full((2 * IN_CH, W)), _full((1, W)),
            _full((W, W)), _full((1, W)),
            _full((W, W)), _full((1, W)),
            _full((W, W)),
        ],
        out_specs=[pl.BlockSpec((NB2, W), lambda i: (i, 0))] * 2,
        out_shape=[jax.ShapeDtypeStruct((NH, W), _F32)] * 2,
    )(x2, w1, b1, w2, b2, w3, b3, w1h)


def _edge_body(g_ref, ea_ref, w1e, b1, w2, b2, w3, b3, m_ref):
    z = g_ref[...] + jnp.dot(ea_ref[...], w1e[...], preferred_element_type=_F32) + b1[...]
    z = jnp.maximum(z, 0.0)
    z = jnp.maximum(jnp.dot(z, w2[...], preferred_element_type=_F32) + b2[...], 0.0)
    m_ref[...] = jnp.dot(z, w3[...], preferred_element_type=_F32) + b3[...]


def _tc_edge(g2, ea2, w1e, b1, w2, b2, w3, b3, co=0):
    nblk = g2.shape[0] // EB2
    return pl.pallas_call(
        _edge_body,
        grid=(nblk,),
        in_specs=[
            pl.BlockSpec((EB2, W), lambda i: (i, 0)),
            pl.BlockSpec((EB2, 8), lambda i, _co=co: (i + _co, 0)),
            _full((8, W)), _full((1, W)),
            _full((W, W)), _full((1, W)),
            _full((W, W)), _full((1, W)),
        ],
        out_specs=pl.BlockSpec((EB2, W), lambda i: (i, 0)),
        out_shape=jax.ShapeDtypeStruct((g2.shape[0], W), _F32),
    )(g2, ea2, w1e, b1, w2, b2, w3, b3)


def _upd_body(h_ref, alo_ref, ahi_ref, u1h, u1lo, u1hi, b1, w2, b2, w3, b3,
              w1h, ho_ref, hp_ref):
    h = h_ref[...]
    acc = jnp.dot(h, u1h[...], preferred_element_type=_F32) + b1[...]
    for f in range(NCH):
        acc = acc + jnp.dot(alo_ref[f], u1lo[f], preferred_element_type=_F32)
        acc = acc + jnp.dot(ahi_ref[f], u1hi[f], preferred_element_type=_F32)
    z = jnp.maximum(acc, 0.0)
    z = jnp.maximum(jnp.dot(z, w2[...], preferred_element_type=_F32) + b2[...], 0.0)
    ho = h + jnp.dot(z, w3[...], preferred_element_type=_F32) + b3[...]
    ho_ref[...] = ho
    hp_ref[...] = jnp.dot(ho, w1h[...], preferred_element_type=_F32)


def _tc_update(h2, agg, u1h, u1lo, u1hi, b1, w2, b2, w3, b3, w1h):
    return pl.pallas_call(
        _upd_body,
        grid=(NH // NB2,),
        in_specs=[
            pl.BlockSpec((NB2, W), lambda i: (i, 0)),
            pl.BlockSpec((NCH, NB2, FCH), lambda i: (0, i, 0)),
            pl.BlockSpec((NCH, NB2, FCH), lambda i: (0, i + NH // NB2, 0)),
            _full((W, W)), _full((NCH, FCH, W)), _full((NCH, FCH, W)),
            _full((1, W)),
            _full((W, W)), _full((1, W)),
            _full((W, W)), _full((1, W)),
            _full((W, W)),
        ],
        out_specs=[pl.BlockSpec((NB2, W), lambda i: (i, 0))] * 2,
        out_shape=[jax.ShapeDtypeStruct((NH, W), _F32)] * 2,
    )(h2, agg, agg, u1h, u1lo, u1hi, b1, w2, b2, w3, b3, w1h)


def _sc_gather(tab, src2d):
    nt = src2d.shape[0]
    nth = nt // 2
    ehc = nt * EB // 2
    mesh = plsc.VectorSubcoreMesh(core_axis_name="c", subcore_axis_name="s")
    per_w = nt // NW
    grp = per_w // NBUF

    @functools.partial(
        pl.kernel,
        mesh=mesh,
        out_type=jax.ShapeDtypeStruct((ehc, W), _F32),
        scratch_types=[pltpu.VMEM((NBUF, EB), jnp.int32),
                       pltpu.VMEM((NBUF * EB, HID), _F32)]
        + [pltpu.SemaphoreType.DMA] * 3,
        compiler_params=pltpu.CompilerParams(use_tc_tiling_on_sc=False),
    )
    def k(tab_hbm, src_hbm, g_hbm, idx_v, rows_v, isem, gsem, wsem):
        wid = lax.axis_index("s") * NC + lax.axis_index("c")
        h = wid // (NW // 2)
        base = wid * per_w

        def body(o, carry):
            t0 = base + o * NBUF
            jh0 = t0 - h * nth
            pltpu.async_copy(src_hbm.at[pl.ds(t0, NBUF), :], idx_v,
                             isem).wait()
            ds = [pltpu.async_copy(tab_hbm.at[idx_v.at[b]],
                                   rows_v.at[pl.ds(b * EB, EB), :], gsem)
                  for b in range(NBUF)]
            for d in ds:
                d.wait()
            pltpu.async_copy(
                rows_v,
                g_hbm.at[pl.ds(jh0 * EB, NBUF * EB), pl.ds(h * HID, HID)],
                wsem).wait()
            return carry

        lax.fori_loop(0, grp, body, 0)

    return k(tab, src2d)


def _sc_scatter(m2, dst2d):
    nt = dst2d.shape[0]
    nth = nt // 2
    mesh = plsc.VectorSubcoreMesh(core_axis_name="c", subcore_axis_name="s")
    per_t = nt // NS
    SB = 8 if per_t % 8 == 0 else 5
    grp = per_t // SB

    @functools.partial(
        pl.kernel,
        mesh=mesh,
        out_type=jax.ShapeDtypeStruct((NCH, N, FCH), _F32),
        scratch_types=[pltpu.VMEM((SB, EB), jnp.int32),
                       pltpu.VMEM((SB * EB, FCH), _F32),
                       pltpu.VMEM((ZR, FCH), _F32), pltpu.VMEM((ZR, FCH), _F32),
                       pltpu.VMEM_SHARED((N, FCH), _F32)]
        + [pltpu.SemaphoreType.DMA] * 2,
        compiler_params=pltpu.CompilerParams(use_tc_tiling_on_sc=False),
    )
    def k(m_hbm, dst_hbm, agg_hbm, idx_v, rows_v, zbuf_v, bounce_v, acc_sh,
          lsem, asem):
        c = lax.axis_index("c")
        s = lax.axis_index("s")
        row0 = s * (N // NS)
        h = s // (NS // 2)

        def zfill(i, carry):
            zbuf_v[i, :] = jnp.zeros((FCH,), _F32)
            return carry

        lax.fori_loop(0, ZR, zfill, 0)

        for q in range(NCH // NC):
            ch = c * (NCH // NC) + q

            def zcopy(i, carry):
                pltpu.sync_copy(zbuf_v, acc_sh.at[pl.ds(row0 + i * ZR, ZR), :])
                return carry

            lax.fori_loop(0, (N // NS) // ZR, zcopy, 0)
            plsc.subcore_barrier()

            def body(o, carry):
                t0 = s * per_t + o * SB
                jh0 = t0 - h * nth
                d1 = pltpu.async_copy(dst_hbm.at[pl.ds(t0, SB), :], idx_v,
                                      lsem)
                d2 = pltpu.async_copy(
                    m_hbm.at[pl.ds(jh0 * EB, SB * EB),
                             pl.ds(h * HID + ch * FCH, FCH)],
                    rows_v, lsem)
                d1.wait()
                d2.wait()
                ds = [pltpu.async_copy(rows_v.at[pl.ds(b * EB, EB), :],
                                       acc_sh.at[idx_v.at[b]], asem,
                                       add=True)
                      for b in range(SB)]
                for d in ds:
                    d.wait()
                return carry

            lax.fori_loop(0, grp, body, 0)
            plsc.subcore_barrier()

            def flush(i, carry):
                r0_ = row0 + i * ZR
                pltpu.sync_copy(acc_sh.at[pl.ds(r0_, ZR), :], bounce_v)
                pltpu.sync_copy(bounce_v, agg_hbm.at[ch, pl.ds(r0_, ZR), :])
                return carry

            lax.fori_loop(0, (N // NS) // ZR, flush, 0)
            plsc.subcore_barrier()

    return k(m2, dst2d)


def kernel(x, ei, ea,
           enc_W1, enc_b1, enc_W2, enc_b2, enc_W3, enc_b3,
           edge_W1, edge_b1, edge_W2, edge_b2, edge_W3, edge_b3,
           upd_W1, upd_b1, upd_W2, upd_b2, upd_W3, upd_b3):
    src = ei[0]
    dst = ei[1]
    src_r = jnp.where(src < NH, 2 * src, 2 * (src - NH) + 1).astype(jnp.int32)
    src2d = src_r.reshape(NT, EB)
    dst2d = dst.reshape(NT, EB)
    nck = NTH // 2
    src_c = [jnp.concatenate([src2d[c * nck:(c + 1) * nck],
                              src2d[NTH + c * nck:NTH + (c + 1) * nck]])
             for c in range(2)]
    dst_c = [jnp.concatenate([dst2d[c * nck:(c + 1) * nck],
                              dst2d[NTH + c * nck:NTH + (c + 1) * nck]])
             for c in range(2)]

    x2 = jnp.concatenate([x[:NH], x[NH:]], axis=1)
    ea2 = jnp.concatenate([ea[:EH], ea[EH:]], axis=1)

    ew1 = _bd(enc_W1)
    ew2 = _bd(enc_W2)
    ew3 = _bd(enc_W3)
    eb1, eb2, eb3 = _bt(enc_b1), _bt(enc_b2), _bt(enc_b3)

    w1h = _bd(edge_W1[:HID])
    w1e = _bd(edge_W1[HID:])
    gw2 = _bd(edge_W2)
    gw3 = _bd(edge_W3)
    gb1, gb2, gb3 = _bt(edge_b1), _bt(edge_b2), _bt(edge_b3)

    u1h = _bd(upd_W1[:HID])
    u1a = upd_W1[HID:].reshape(NCH, FCH, HID)
    u1lo = jnp.zeros((NCH, FCH, W), _F32).at[:, :, :HID].set(u1a)
    u1hi = jnp.zeros((NCH, FCH, W), _F32).at[:, :, HID:].set(u1a)
    uw2 = _bd(upd_W2)
    uw3 = _bd(upd_W3)
    ub1, ub2, ub3 = _bt(upd_b1), _bt(upd_b2), _bt(upd_b3)

    h2, hp2 = _tc_encoder(x2, ew1, eb1, ew2, eb2, ew3, eb3, w1h)
    ecb = (EH // 2) // EB2
    for _ in range(MP):
        tab = hp2.reshape(N, HID)
        gs = [_sc_gather(tab, src_c[c]) for c in range(2)]
        ms = [_tc_edge(gs[c], ea2, w1e, gb1, gw2, gb2, gw3, gb3, co=c * ecb)
              for c in range(2)]
        aggs = [_sc_scatter(ms[c], dst_c[c]) for c in range(2)]
        agg = aggs[0] + aggs[1]
        h2, hp2 = _tc_update(h2, agg, u1h, u1lo, u1hi, ub1, uw2, ub2, uw3,
                             ub3, w1h)
    return jnp.concatenate([h2[:, :HID], h2[:, HID:]], axis=0)


# speedup vs baseline: 1.0189x; 1.0057x over previous
import functools

import jax
import jax.numpy as jnp
from jax import lax
from jax.experimental import pallas as pl
from jax.experimental.pallas import tpu as pltpu
from jax.experimental.pallas import tpu_sc as plsc

N = 100000
E = 1600000
NH = N // 2
EH = E // 2
IN_CH = 6
HID = 64
W = 2 * HID
MP = 2

FCH = 16
NCH = HID // FCH
EB = 100
NT = E // EB
NTH = EH // EB
NC = 2
NS = 16
NW = NC * NS
NBUF = 5
ZR = 250

NB2 = 2000
EB2 = 4000

_F32 = jnp.float32


def _full(shape):
    return pl.BlockSpec(shape, lambda i: tuple(0 for _ in shape))


def _bd(w):
    fi, fo = w.shape
    z = jnp.zeros((2 * fi, 2 * fo), w.dtype)
    return z.at[:fi, :fo].set(w).at[fi:, fo:].set(w)


def _bt(b):
    return jnp.concatenate([b, b]).reshape(1, W)


def _enc_body(x_ref, w1, b1, w2, b2, w3, b3, w1h, h_ref, hp_ref):
    z = jnp.maximum(jnp.dot(x_ref[...], w1[...], preferred_element_type=_F32) + b1[...], 0.0)
    z = jnp.maximum(jnp.dot(z, w2[...], preferred_element_type=_F32) + b2[...], 0.0)
    h = jnp.dot(z, w3[...], preferred_element_type=_F32) + b3[...]
    h_ref[...] = h
    hp_ref[...] = jnp.dot(h, w1h[...], preferred_element_type=_F32)


def _tc_encoder(x2, w1, b1, w2, b2, w3, b3, w1h):
    return pl.pallas_call(
        _enc_body,
        grid=(NH // NB2,),
        in_specs=[
            pl.BlockSpec((NB2, 2 * IN_CH), lambda i: (i, 0)),
            _full((2 * IN_CH, W)), _full((1, W)),
            _full((W, W)), _full((1, W)),
            _full((W, W)), _full((1, W)),
            _full((W, W)),
        ],
        out_specs=[pl.BlockSpec((NB2, W), lambda i: (i, 0))] * 2,
        out_shape=[jax.ShapeDtypeStruct((NH, W), _F32)] * 2,
    )(x2, w1, b1, w2, b2, w3, b3, w1h)


def _edge_body(g_ref, ea_ref, w1e, b1, w2, b2, w3, b3, m_ref):
    z = g_ref[...] + jnp.dot(ea_ref[...], w1e[...], preferred_element_type=_F32) + b1[...]
    z = jnp.maximum(z, 0.0)
    z = jnp.maximum(jnp.dot(z, w2[...], preferred_element_type=_F32) + b2[...], 0.0)
    m_ref[...] = jnp.dot(z, w3[...], preferred_element_type=_F32) + b3[...]


def _tc_edge(g2, ea2, w1e, b1, w2, b2, w3, b3, co=0):
    nblk = g2.shape[0] // EB2
    return pl.pallas_call(
        _edge_body,
        grid=(nblk,),
        in_specs=[
            pl.BlockSpec((EB2, W), lambda i: (i, 0)),
            pl.BlockSpec((EB2, 8), lambda i, _co=co: (i + _co, 0)),
            _full((8, W)), _full((1, W)),
            _full((W, W)), _full((1, W)),
            _full((W, W)), _full((1, W)),
        ],
        out_specs=pl.BlockSpec((EB2, W), lambda i: (i, 0)),
        out_shape=jax.ShapeDtypeStruct((g2.shape[0], W), _F32),
    )(g2, ea2, w1e, b1, w2, b2, w3, b3)


def _upd_body(h_ref, alo_ref, ahi_ref, u1h, u1lo, u1hi, b1, w2, b2, w3, b3,
              w1h, ho_ref, hp_ref):
    h = h_ref[...]
    acc = jnp.dot(h, u1h[...], preferred_element_type=_F32) + b1[...]
    for f in range(NCH):
        acc = acc + jnp.dot(alo_ref[f], u1lo[f], preferred_element_type=_F32)
        acc = acc + jnp.dot(ahi_ref[f], u1hi[f], preferred_element_type=_F32)
    z = jnp.maximum(acc, 0.0)
    z = jnp.maximum(jnp.dot(z, w2[...], preferred_element_type=_F32) + b2[...], 0.0)
    ho = h + jnp.dot(z, w3[...], preferred_element_type=_F32) + b3[...]
    ho_ref[...] = ho
    hp_ref[...] = jnp.dot(ho, w1h[...], preferred_element_type=_F32)


def _tc_update(h2, agg, u1h, u1lo, u1hi, b1, w2, b2, w3, b3, w1h):
    return pl.pallas_call(
        _upd_body,
        grid=(NH // NB2,),
        in_specs=[
            pl.BlockSpec((NB2, W), lambda i: (i, 0)),
            pl.BlockSpec((NCH, NB2, FCH), lambda i: (0, i, 0)),
            pl.BlockSpec((NCH, NB2, FCH), lambda i: (0, i + NH // NB2, 0)),
            _full((W, W)), _full((NCH, FCH, W)), _full((NCH, FCH, W)),
            _full((1, W)),
            _full((W, W)), _full((1, W)),
            _full((W, W)), _full((1, W)),
            _full((W, W)),
        ],
        out_specs=[pl.BlockSpec((NB2, W), lambda i: (i, 0))] * 2,
        out_shape=[jax.ShapeDtypeStruct((NH, W), _F32)] * 2,
    )(h2, agg, agg, u1h, u1lo, u1hi, b1, w2, b2, w3, b3, w1h)


def _sc_gather(tab, src2d):
    nt = src2d.shape[0]
    nth = nt // 2
    ehc = nt * EB // 2
    mesh = plsc.VectorSubcoreMesh(core_axis_name="c", subcore_axis_name="s")
    per_w = nt // NW
    grp = per_w // NBUF

    @functools.partial(
        pl.kernel,
        mesh=mesh,
        out_type=jax.ShapeDtypeStruct((ehc, W), _F32),
        scratch_types=[pltpu.VMEM((NBUF, EB), jnp.int32),
                       pltpu.VMEM((NBUF * EB, HID), _F32)]
        + [pltpu.SemaphoreType.DMA] * 3,
        compiler_params=pltpu.CompilerParams(use_tc_tiling_on_sc=False),
    )
    def k(tab_hbm, src_hbm, g_hbm, idx_v, rows_v, isem, gsem, wsem):
        wid = lax.axis_index("s") * NC + lax.axis_index("c")
        h = wid // (NW // 2)
        base = wid * per_w

        def body(o, carry):
            t0 = base + o * NBUF
            jh0 = t0 - h * nth
            pltpu.async_copy(src_hbm.at[pl.ds(t0, NBUF), :], idx_v,
                             isem).wait()
            ds = [pltpu.async_copy(tab_hbm.at[idx_v.at[b]],
                                   rows_v.at[pl.ds(b * EB, EB), :], gsem)
                  for b in range(NBUF)]
            for d in ds:
                d.wait()
            pltpu.async_copy(
                rows_v,
                g_hbm.at[pl.ds(jh0 * EB, NBUF * EB), pl.ds(h * HID, HID)],
                wsem).wait()
            return carry

        lax.fori_loop(0, grp, body, 0)

    return k(tab, src2d)


def _sc_scatter(m2, dst2d):
    nt = dst2d.shape[0]
    nth = nt // 2
    mesh = plsc.VectorSubcoreMesh(core_axis_name="c", subcore_axis_name="s")
    per_t = nt // NS
    SB = 8 if per_t % 8 == 0 else 5
    grp = per_t // SB

    @functools.partial(
        pl.kernel,
        mesh=mesh,
        out_type=jax.ShapeDtypeStruct((NCH, N, FCH), _F32),
        scratch_types=[pltpu.VMEM((SB, EB), jnp.int32),
                       pltpu.VMEM((SB * EB, FCH), _F32),
                       pltpu.VMEM((ZR, FCH), _F32),
                       pltpu.VMEM_SHARED((N, FCH), _F32)]
        + [pltpu.SemaphoreType.DMA] * 2,
        compiler_params=pltpu.CompilerParams(use_tc_tiling_on_sc=False),
    )
    def k(m_hbm, dst_hbm, agg_hbm, idx_v, rows_v, zbuf_v, acc_sh,
          lsem, asem):
        c = lax.axis_index("c")
        s = lax.axis_index("s")
        row0 = s * (N // NS)
        h = s // (NS // 2)

        def zfill(i, carry):
            zbuf_v[i, :] = jnp.zeros((FCH,), _F32)
            return carry

        lax.fori_loop(0, ZR, zfill, 0)

        for q in range(NCH // NC):
            ch = c * (NCH // NC) + q

            zds = [pltpu.async_copy(
                       zbuf_v, acc_sh.at[pl.ds(row0 + i * ZR, ZR), :], lsem)
                   for i in range((N // NS) // ZR)]
            for d in zds:
                d.wait()
            plsc.subcore_barrier()

            def body(o, carry):
                t0 = s * per_t + o * SB
                jh0 = t0 - h * nth
                d1 = pltpu.async_copy(dst_hbm.at[pl.ds(t0, SB), :], idx_v,
                                      lsem)
                d2 = pltpu.async_copy(
                    m_hbm.at[pl.ds(jh0 * EB, SB * EB),
                             pl.ds(h * HID + ch * FCH, FCH)],
                    rows_v, lsem)
                d1.wait()
                d2.wait()
                ds = [pltpu.async_copy(rows_v.at[pl.ds(b * EB, EB), :],
                                       acc_sh.at[idx_v.at[b]], asem,
                                       add=True)
                      for b in range(SB)]
                for d in ds:
                    d.wait()
                return carry

            lax.fori_loop(0, grp, body, 0)
            plsc.subcore_barrier()

            pltpu.sync_copy(acc_sh.at[pl.ds(row0, N // NS), :],
                            agg_hbm.at[ch, pl.ds(row0, N // NS), :])
            plsc.subcore_barrier()

    return k(m2, dst2d)


def kernel(x, ei, ea,
           enc_W1, enc_b1, enc_W2, enc_b2, enc_W3, enc_b3,
           edge_W1, edge_b1, edge_W2, edge_b2, edge_W3, edge_b3,
           upd_W1, upd_b1, upd_W2, upd_b2, upd_W3, upd_b3):
    src = ei[0]
    dst = ei[1]
    src_r = jnp.where(src < NH, 2 * src, 2 * (src - NH) + 1).astype(jnp.int32)
    src2d = src_r.reshape(NT, EB)
    dst2d = dst.reshape(NT, EB)
    nck = NTH // 2
    src_c = [jnp.concatenate([src2d[c * nck:(c + 1) * nck],
                              src2d[NTH + c * nck:NTH + (c + 1) * nck]])
             for c in range(2)]
    dst_c = [jnp.concatenate([dst2d[c * nck:(c + 1) * nck],
                              dst2d[NTH + c * nck:NTH + (c + 1) * nck]])
             for c in range(2)]

    x2 = jnp.concatenate([x[:NH], x[NH:]], axis=1)
    ea2 = jnp.concatenate([ea[:EH], ea[EH:]], axis=1)

    ew1 = _bd(enc_W1)
    ew2 = _bd(enc_W2)
    ew3 = _bd(enc_W3)
    eb1, eb2, eb3 = _bt(enc_b1), _bt(enc_b2), _bt(enc_b3)

    w1h = _bd(edge_W1[:HID])
    w1e = _bd(edge_W1[HID:])
    gw2 = _bd(edge_W2)
    gw3 = _bd(edge_W3)
    gb1, gb2, gb3 = _bt(edge_b1), _bt(edge_b2), _bt(edge_b3)

    u1h = _bd(upd_W1[:HID])
    u1a = upd_W1[HID:].reshape(NCH, FCH, HID)
    u1lo = jnp.zeros((NCH, FCH, W), _F32).at[:, :, :HID].set(u1a)
    u1hi = jnp.zeros((NCH, FCH, W), _F32).at[:, :, HID:].set(u1a)
    uw2 = _bd(upd_W2)
    uw3 = _bd(upd_W3)
    ub1, ub2, ub3 = _bt(upd_b1), _bt(upd_b2), _bt(upd_b3)

    h2, hp2 = _tc_encoder(x2, ew1, eb1, ew2, eb2, ew3, eb3, w1h)
    ecb = (EH // 2) // EB2
    for _ in range(MP):
        tab = hp2.reshape(N, HID)
        gs = [_sc_gather(tab, src_c[c]) for c in range(2)]
        ms = [_tc_edge(gs[c], ea2, w1e, gb1, gw2, gb2, gw3, gb3, co=c * ecb)
              for c in range(2)]
        aggs = [_sc_scatter(ms[c], dst_c[c]) for c in range(2)]
        agg = aggs[0] + aggs[1]
        h2, hp2 = _tc_update(h2, agg, u1h, u1lo, u1hi, ub1, uw2, ub2, uw3,
                             ub3, w1h)
    return jnp.concatenate([h2[:, :HID], h2[:, HID:]], axis=0)


# speedup vs baseline: 1.1311x; 1.1102x over previous
import functools

import jax
import jax.numpy as jnp
from jax import lax
from jax.experimental import pallas as pl
from jax.experimental.pallas import tpu as pltpu
from jax.experimental.pallas import tpu_sc as plsc

N = 100000
E = 1600000
NH = N // 2
EH = E // 2
IN_CH = 6
HID = 64
W = 2 * HID
MP = 2

FCH = 16
NCH = HID // FCH
EB = 100
NT = E // EB
NTH = EH // EB
NC = 2
NS = 16
NW = NC * NS
NBUF = 5
ZR = 250

NB2 = 2000
EB2 = 4000

_F32 = jnp.float32


def _full(shape):
    return pl.BlockSpec(shape, lambda i: tuple(0 for _ in shape))


def _bd(w):
    fi, fo = w.shape
    z = jnp.zeros((2 * fi, 2 * fo), w.dtype)
    return z.at[:fi, :fo].set(w).at[fi:, fo:].set(w)


def _bt(b):
    return jnp.concatenate([b, b]).reshape(1, W)


def _enc_body(x_ref, w1, b1, w2, b2, w3, b3, w1h, h_ref, hp_ref):
    z = jnp.maximum(jnp.dot(x_ref[...], w1[...], preferred_element_type=_F32) + b1[...], 0.0)
    z = jnp.maximum(jnp.dot(z, w2[...], preferred_element_type=_F32) + b2[...], 0.0)
    h = jnp.dot(z, w3[...], preferred_element_type=_F32) + b3[...]
    h_ref[...] = h
    hp_ref[...] = jnp.dot(h, w1h[...], preferred_element_type=_F32)


def _tc_encoder(x2, w1, b1, w2, b2, w3, b3, w1h):
    return pl.pallas_call(
        _enc_body,
        grid=(NH // NB2,),
        in_specs=[
            pl.BlockSpec((NB2, 2 * IN_CH), lambda i: (i, 0)),
            _full((2 * IN_CH, W)), _full((1, W)),
            _full((W, W)), _full((1, W)),
            _full((W, W)), _full((1, W)),
            _full((W, W)),
        ],
        out_specs=[pl.BlockSpec((NB2, W), lambda i: (i, 0))] * 2,
        out_shape=[jax.ShapeDtypeStruct((NH, W), _F32)] * 2,
    )(x2, w1, b1, w2, b2, w3, b3, w1h)


def _edge_body(g_ref, ea_ref, w1e, b1, w2, b2, w3, b3, m_ref):
    z = g_ref[...] + jnp.dot(ea_ref[...], w1e[...], preferred_element_type=_F32) + b1[...]
    z = jnp.maximum(z, 0.0)
    z = jnp.maximum(jnp.dot(z, w2[...], preferred_element_type=_F32) + b2[...], 0.0)
    m_ref[...] = jnp.dot(z, w3[...], preferred_element_type=_F32) + b3[...]


def _tc_edge(g2, ea2, w1e, b1, w2, b2, w3, b3, co=0):
    nblk = g2.shape[0] // EB2
    return pl.pallas_call(
        _edge_body,
        grid=(nblk,),
        in_specs=[
            pl.BlockSpec((EB2, W), lambda i: (i, 0)),
            pl.BlockSpec((EB2, 8), lambda i, _co=co: (i + _co, 0)),
            _full((8, W)), _full((1, W)),
            _full((W, W)), _full((1, W)),
            _full((W, W)), _full((1, W)),
        ],
        out_specs=pl.BlockSpec((EB2, W), lambda i: (i, 0)),
        out_shape=jax.ShapeDtypeStruct((g2.shape[0], W), _F32),
    )(g2, ea2, w1e, b1, w2, b2, w3, b3)


def _upd_body(h_ref, alo_ref, ahi_ref, u1h, u1lo, u1hi, b1, w2, b2, w3, b3,
              w1h, ho_ref, hp_ref):
    h = h_ref[...]
    acc = jnp.dot(h, u1h[...], preferred_element_type=_F32) + b1[...]
    for f in range(NCH):
        acc = acc + jnp.dot(alo_ref[f], u1lo[f], preferred_element_type=_F32)
        acc = acc + jnp.dot(ahi_ref[f], u1hi[f], preferred_element_type=_F32)
    z = jnp.maximum(acc, 0.0)
    z = jnp.maximum(jnp.dot(z, w2[...], preferred_element_type=_F32) + b2[...], 0.0)
    ho = h + jnp.dot(z, w3[...], preferred_element_type=_F32) + b3[...]
    ho_ref[...] = ho
    hp_ref[...] = jnp.dot(ho, w1h[...], preferred_element_type=_F32)


def _tc_update(h2, agg, u1h, u1lo, u1hi, b1, w2, b2, w3, b3, w1h):
    return pl.pallas_call(
        _upd_body,
        grid=(NH // NB2,),
        in_specs=[
            pl.BlockSpec((NB2, W), lambda i: (i, 0)),
            pl.BlockSpec((NCH, NB2, FCH), lambda i: (0, i, 0)),
            pl.BlockSpec((NCH, NB2, FCH), lambda i: (0, i + NH // NB2, 0)),
            _full((W, W)), _full((NCH, FCH, W)), _full((NCH, FCH, W)),
            _full((1, W)),
            _full((W, W)), _full((1, W)),
            _full((W, W)), _full((1, W)),
            _full((W, W)),
        ],
        out_specs=[pl.BlockSpec((NB2, W), lambda i: (i, 0))] * 2,
        out_shape=[jax.ShapeDtypeStruct((NH, W), _F32)] * 2,
    )(h2, agg, agg, u1h, u1lo, u1hi, b1, w2, b2, w3, b3, w1h)


def _sc_gather(tab, src2d):
    nt = src2d.shape[0]
    nth = nt // 2
    ehc = nt * EB // 2
    mesh = plsc.VectorSubcoreMesh(core_axis_name="c", subcore_axis_name="s")
    per_w = nt // NW
    grp = per_w // NBUF

    @functools.partial(
        pl.kernel,
        mesh=mesh,
        out_type=jax.ShapeDtypeStruct((ehc, W), _F32),
        scratch_types=[pltpu.VMEM((NBUF, EB), jnp.int32),
                       pltpu.VMEM((NBUF * EB, HID), _F32)]
        + [pltpu.SemaphoreType.DMA] * 3,
        compiler_params=pltpu.CompilerParams(use_tc_tiling_on_sc=False),
    )
    def k(tab_hbm, src_hbm, g_hbm, idx_v, rows_v, isem, gsem, wsem):
        wid = lax.axis_index("s") * NC + lax.axis_index("c")
        h = wid // (NW // 2)
        base = wid * per_w

        def body(o, carry):
            t0 = base + o * NBUF
            jh0 = t0 - h * nth
            pltpu.async_copy(src_hbm.at[pl.ds(t0, NBUF), :], idx_v,
                             isem).wait()
            ds = [pltpu.async_copy(tab_hbm.at[idx_v.at[b]],
                                   rows_v.at[pl.ds(b * EB, EB), :], gsem)
                  for b in range(NBUF)]
            for d in ds:
                d.wait()
            pltpu.async_copy(
                rows_v,
                g_hbm.at[pl.ds(jh0 * EB, NBUF * EB), pl.ds(h * HID, HID)],
                wsem).wait()
            return carry

        lax.fori_loop(0, grp, body, 0)

    return k(tab, src2d)


def _sc_scatter(m2, dst2d):
    nt = dst2d.shape[0]
    nth = nt // 2
    mesh = plsc.VectorSubcoreMesh(core_axis_name="c", subcore_axis_name="s")
    per_t = nt // NS
    SB = 8 if per_t % 8 == 0 else 5
    grp = per_t // SB

    @functools.partial(
        pl.kernel,
        mesh=mesh,
        out_type=jax.ShapeDtypeStruct((NCH, N, FCH), _F32),
        scratch_types=[pltpu.VMEM((SB, EB), jnp.int32),
                       pltpu.VMEM((SB * EB, FCH), _F32),
                       pltpu.VMEM((ZR, FCH), _F32),
                       pltpu.VMEM_SHARED((N, FCH), _F32)]
        + [pltpu.SemaphoreType.DMA] * 2,
        compiler_params=pltpu.CompilerParams(use_tc_tiling_on_sc=False),
    )
    def k(m_hbm, dst_hbm, agg_hbm, idx_v, rows_v, zbuf_v, acc_sh,
          lsem, asem):
        c = lax.axis_index("c")
        s = lax.axis_index("s")
        row0 = s * (N // NS)
        h = s // (NS // 2)

        def zfill(i, carry):
            zbuf_v[i, :] = jnp.zeros((FCH,), _F32)
            return carry

        lax.fori_loop(0, ZR, zfill, 0)

        for q in range(NCH // NC):
            ch = c * (NCH // NC) + q

            zds = [pltpu.async_copy(
                       zbuf_v, acc_sh.at[pl.ds(row0 + i * ZR, ZR), :], lsem)
                   for i in range((N // NS) // ZR)]
            for d in zds:
                d.wait()
            plsc.subcore_barrier()

            def body(o, carry):
                t0 = s * per_t + o * SB
                jh0 = t0 - h * nth
                d1 = pltpu.async_copy(dst_hbm.at[pl.ds(t0, SB), :], idx_v,
                                      lsem)
                d2 = pltpu.async_copy(
                    m_hbm.at[pl.ds(jh0 * EB, SB * EB),
                             pl.ds(h * HID + ch * FCH, FCH)],
                    rows_v, lsem)
                d1.wait()
                d2.wait()
                ds = [pltpu.async_copy(rows_v.at[pl.ds(b * EB, EB), :],
                                       acc_sh.at[idx_v.at[b]], asem,
                                       add=True)
                      for b in range(SB)]
                for d in ds:
                    d.wait()
                return carry

            lax.fori_loop(0, grp, body, 0)
            plsc.subcore_barrier()

            pltpu.sync_copy(acc_sh.at[pl.ds(row0, N // NS), :],
                            agg_hbm.at[ch, pl.ds(row0, N // NS), :])
            plsc.subcore_barrier()

    return k(m2, dst2d)


def kernel(x, ei, ea,
           enc_W1, enc_b1, enc_W2, enc_b2, enc_W3, enc_b3,
           edge_W1, edge_b1, edge_W2, edge_b2, edge_W3, edge_b3,
           upd_W1, upd_b1, upd_W2, upd_b2, upd_W3, upd_b3):
    src = ei[0]
    dst = ei[1]
    src_r = jnp.where(src < NH, 2 * src, 2 * (src - NH) + 1).astype(jnp.int32)
    src2d = src_r.reshape(NT, EB)
    dst2d = dst.reshape(NT, EB)

    x2 = jnp.concatenate([x[:NH], x[NH:]], axis=1)
    ea2 = jnp.concatenate([ea[:EH], ea[EH:]], axis=1)

    ew1 = _bd(enc_W1)
    ew2 = _bd(enc_W2)
    ew3 = _bd(enc_W3)
    eb1, eb2, eb3 = _bt(enc_b1), _bt(enc_b2), _bt(enc_b3)

    w1h = _bd(edge_W1[:HID])
    w1e = _bd(edge_W1[HID:])
    gw2 = _bd(edge_W2)
    gw3 = _bd(edge_W3)
    gb1, gb2, gb3 = _bt(edge_b1), _bt(edge_b2), _bt(edge_b3)

    u1h = _bd(upd_W1[:HID])
    u1a = upd_W1[HID:].reshape(NCH, FCH, HID)
    u1lo = jnp.zeros((NCH, FCH, W), _F32).at[:, :, :HID].set(u1a)
    u1hi = jnp.zeros((NCH, FCH, W), _F32).at[:, :, HID:].set(u1a)
    uw2 = _bd(upd_W2)
    uw3 = _bd(upd_W3)
    ub1, ub2, ub3 = _bt(upd_b1), _bt(upd_b2), _bt(upd_b3)

    h2, hp2 = _tc_encoder(x2, ew1, eb1, ew2, eb2, ew3, eb3, w1h)
    for _ in range(MP):
        g2 = _sc_gather(hp2.reshape(N, HID), src2d)
        m2 = _tc_edge(g2, ea2, w1e, gb1, gw2, gb2, gw3, gb3)
        agg = _sc_scatter(m2, dst2d)
        h2, hp2 = _tc_update(h2, agg, u1h, u1lo, u1hi, ub1, uw2, ub2, uw3,
                             ub3, w1h)
    return jnp.concatenate([h2[:, :HID], h2[:, HID:]], axis=0)


# speedup vs baseline: 1.2012x; 1.0619x over previous
import functools

import jax
import jax.numpy as jnp
from jax import lax
from jax.experimental import pallas as pl
from jax.experimental.pallas import tpu as pltpu
from jax.experimental.pallas import tpu_sc as plsc

N = 100000
E = 1600000
NH = N // 2
EH = E // 2
IN_CH = 6
HID = 64
W = 2 * HID
MP = 2

FCH = 16
NCH = HID // FCH
EB = 100
NT = E // EB
NTH = EH // EB
NC = 2
NS = 16
NW = NC * NS
NBUF = 5
ZR = 250

NB2 = 2000
EB2 = 4000

_F32 = jnp.float32


def _full(shape):
    return pl.BlockSpec(shape, lambda i: tuple(0 for _ in shape))


def _bd(w):
    fi, fo = w.shape
    z = jnp.zeros((2 * fi, 2 * fo), w.dtype)
    return z.at[:fi, :fo].set(w).at[fi:, fo:].set(w)


def _bt(b):
    return jnp.concatenate([b, b]).reshape(1, W)


def _enc_body(x_ref, w1, b1, w2, b2, w3, b3, w1h, h_ref, hp_ref):
    z = jnp.maximum(jnp.dot(x_ref[...], w1[...], preferred_element_type=_F32) + b1[...], 0.0)
    z = jnp.maximum(jnp.dot(z, w2[...], preferred_element_type=_F32) + b2[...], 0.0)
    h = jnp.dot(z, w3[...], preferred_element_type=_F32) + b3[...]
    h_ref[...] = h
    hp_ref[...] = jnp.dot(h, w1h[...], preferred_element_type=_F32)


def _tc_encoder(x2, w1, b1, w2, b2, w3, b3, w1h):
    return pl.pallas_call(
        _enc_body,
        grid=(NH // NB2,),
        in_specs=[
            pl.BlockSpec((NB2, 2 * IN_CH), lambda i: (i, 0)),
            _full((2 * IN_CH, W)), _full((1, W)),
            _full((W, W)), _full((1, W)),
            _full((W, W)), _full((1, W)),
            _full((W, W)),
        ],
        out_specs=[pl.BlockSpec((NB2, W), lambda i: (i, 0))] * 2,
        out_shape=[jax.ShapeDtypeStruct((NH, W), _F32)] * 2,
    )(x2, w1, b1, w2, b2, w3, b3, w1h)


def _edge_body(g_ref, ea_ref, w1e, b1, w2, b2, w3, b3, m_ref):
    z = g_ref[...] + jnp.dot(ea_ref[...], w1e[...], preferred_element_type=_F32) + b1[...]
    z = jnp.maximum(z, 0.0)
    z = jnp.maximum(jnp.dot(z, w2[...], preferred_element_type=_F32) + b2[...], 0.0)
    m_ref[...] = jnp.dot(z, w3[...], preferred_element_type=_F32) + b3[...]


def _tc_edge(g2, ea2, w1e, b1, w2, b2, w3, b3, co=0):
    nblk = g2.shape[0] // EB2
    return pl.pallas_call(
        _edge_body,
        grid=(nblk,),
        in_specs=[
            pl.BlockSpec((EB2, W), lambda i: (i, 0)),
            pl.BlockSpec((EB2, 8), lambda i, _co=co: (i + _co, 0)),
            _full((8, W)), _full((1, W)),
            _full((W, W)), _full((1, W)),
            _full((W, W)), _full((1, W)),
        ],
        out_specs=pl.BlockSpec((EB2, W), lambda i: (i, 0)),
        out_shape=jax.ShapeDtypeStruct((g2.shape[0], W), _F32),
    )(g2, ea2, w1e, b1, w2, b2, w3, b3)


def _upd_body(h_ref, alo_ref, ahi_ref, u1h, u1lo, u1hi, b1, w2, b2, w3, b3,
              w1h, ho_ref, hp_ref):
    h = h_ref[...]
    acc = jnp.dot(h, u1h[...], preferred_element_type=_F32) + b1[...]
    for f in range(NCH):
        acc = acc + jnp.dot(alo_ref[f], u1lo[f], preferred_element_type=_F32)
        acc = acc + jnp.dot(ahi_ref[f], u1hi[f], preferred_element_type=_F32)
    z = jnp.maximum(acc, 0.0)
    z = jnp.maximum(jnp.dot(z, w2[...], preferred_element_type=_F32) + b2[...], 0.0)
    ho = h + jnp.dot(z, w3[...], preferred_element_type=_F32) + b3[...]
    ho_ref[...] = ho
    hp_ref[...] = jnp.dot(ho, w1h[...], preferred_element_type=_F32)


def _tc_update(h2, agg, u1h, u1lo, u1hi, b1, w2, b2, w3, b3, w1h):
    return pl.pallas_call(
        _upd_body,
        grid=(NH // NB2,),
        in_specs=[
            pl.BlockSpec((NB2, W), lambda i: (i, 0)),
            pl.BlockSpec((NCH, NB2, FCH), lambda i: (0, i, 0)),
            pl.BlockSpec((NCH, NB2, FCH), lambda i: (0, i + NH // NB2, 0)),
            _full((W, W)), _full((NCH, FCH, W)), _full((NCH, FCH, W)),
            _full((1, W)),
            _full((W, W)), _full((1, W)),
            _full((W, W)), _full((1, W)),
            _full((W, W)),
        ],
        out_specs=[pl.BlockSpec((NB2, W), lambda i: (i, 0))] * 2,
        out_shape=[jax.ShapeDtypeStruct((NH, W), _F32)] * 2,
    )(h2, agg, agg, u1h, u1lo, u1hi, b1, w2, b2, w3, b3, w1h)


def _sc_gather(tab, src2d):
    nt = src2d.shape[0]
    nth = nt // 2
    ehc = nt * EB // 2
    mesh = plsc.VectorSubcoreMesh(core_axis_name="c", subcore_axis_name="s")
    per_w = nt // NW
    grp = per_w // NBUF

    @functools.partial(
        pl.kernel,
        mesh=mesh,
        out_type=jax.ShapeDtypeStruct((ehc, W), _F32),
        scratch_types=[pltpu.VMEM((NBUF, EB), jnp.int32),
                       pltpu.VMEM((NBUF * EB, HID), _F32)]
        + [pltpu.SemaphoreType.DMA] * 3,
        compiler_params=pltpu.CompilerParams(use_tc_tiling_on_sc=False),
    )
    def k(tab_hbm, src_hbm, g_hbm, idx_v, rows_v, isem, gsem, wsem):
        wid = lax.axis_index("s") * NC + lax.axis_index("c")
        h = wid // (NW // 2)
        base = wid * per_w

        def body(o, carry):
            t0 = base + o * NBUF
            jh0 = t0 - h * nth
            pltpu.async_copy(src_hbm.at[pl.ds(t0, NBUF), :], idx_v,
                             isem).wait()
            ds = [pltpu.async_copy(tab_hbm.at[idx_v.at[b]],
                                   rows_v.at[pl.ds(b * EB, EB), :], gsem)
                  for b in range(NBUF)]
            for d in ds:
                d.wait()
            pltpu.async_copy(
                rows_v,
                g_hbm.at[pl.ds(jh0 * EB, NBUF * EB), pl.ds(h * HID, HID)],
                wsem).wait()
            return carry

        lax.fori_loop(0, grp, body, 0)

    return k(tab, src2d)


def _sc_scatter(m2, dst2d):
    nt = dst2d.shape[0]
    nth = nt // 2
    mesh = plsc.VectorSubcoreMesh(core_axis_name="c", subcore_axis_name="s")
    per_t = nt // NS
    SB = 4
    grp = per_t // (2 * SB)

    @functools.partial(
        pl.kernel,
        mesh=mesh,
        out_type=jax.ShapeDtypeStruct((NCH, N, FCH), _F32),
        scratch_types=[pltpu.VMEM((2, SB, EB), jnp.int32),
                       pltpu.VMEM((2, SB * EB, FCH), _F32),
                       pltpu.VMEM((ZR, FCH), _F32),
                       pltpu.VMEM_SHARED((N, FCH), _F32)]
        + [pltpu.SemaphoreType.DMA] * 3,
        compiler_params=pltpu.CompilerParams(use_tc_tiling_on_sc=False),
    )
    def k(m_hbm, dst_hbm, agg_hbm, idx_v, rows_v, zbuf_v, acc_sh,
          l0sem, l1sem, asem):
        c = lax.axis_index("c")
        s = lax.axis_index("s")
        row0 = s * (N // NS)
        h = s // (NS // 2)

        def zfill(i, carry):
            zbuf_v[i, :] = jnp.zeros((FCH,), _F32)
            return carry

        lax.fori_loop(0, ZR, zfill, 0)

        for q in range(NCH // NC):
            ch = c * (NCH // NC) + q

            zds = [pltpu.async_copy(
                       zbuf_v, acc_sh.at[pl.ds(row0 + i * ZR, ZR), :], l0sem)
                   for i in range((N // NS) // ZR)]
            for d in zds:
                d.wait()
            plsc.subcore_barrier()

            def body(o, carry):
                lds = []
                for p in range(2):
                    t0 = s * per_t + (2 * o + p) * SB
                    jh0 = t0 - h * nth
                    sem = l0sem if p == 0 else l1sem
                    lds.append((
                        pltpu.async_copy(dst_hbm.at[pl.ds(t0, SB), :],
                                         idx_v.at[p], sem),
                        pltpu.async_copy(
                            m_hbm.at[pl.ds(jh0 * EB, SB * EB),
                                     pl.ds(h * HID + ch * FCH, FCH)],
                            rows_v.at[p], sem)))
                ads = []
                for p in range(2):
                    for d in lds[p]:
                        d.wait()
                    ads.extend(
                        pltpu.async_copy(rows_v.at[p, pl.ds(b * EB, EB), :],
                                         acc_sh.at[idx_v.at[p, b]], asem,
                                         add=True)
                        for b in range(SB))
                for d in ads:
                    d.wait()
                return carry

            lax.fori_loop(0, grp, body, 0)
            plsc.subcore_barrier()

            pltpu.sync_copy(acc_sh.at[pl.ds(row0, N // NS), :],
                            agg_hbm.at[ch, pl.ds(row0, N // NS), :])
            plsc.subcore_barrier()

    return k(m2, dst2d)


def kernel(x, ei, ea,
           enc_W1, enc_b1, enc_W2, enc_b2, enc_W3, enc_b3,
           edge_W1, edge_b1, edge_W2, edge_b2, edge_W3, edge_b3,
           upd_W1, upd_b1, upd_W2, upd_b2, upd_W3, upd_b3):
    src = ei[0]
    dst = ei[1]
    src_r = jnp.where(src < NH, 2 * src, 2 * (src - NH) + 1).astype(jnp.int32)
    src2d = src_r.reshape(NT, EB)
    dst2d = dst.reshape(NT, EB)

    x2 = jnp.concatenate([x[:NH], x[NH:]], axis=1)
    ea2 = jnp.concatenate([ea[:EH], ea[EH:]], axis=1)

    ew1 = _bd(enc_W1)
    ew2 = _bd(enc_W2)
    ew3 = _bd(enc_W3)
    eb1, eb2, eb3 = _bt(enc_b1), _bt(enc_b2), _bt(enc_b3)

    w1h = _bd(edge_W1[:HID])
    w1e = _bd(edge_W1[HID:])
    gw2 = _bd(edge_W2)
    gw3 = _bd(edge_W3)
    gb1, gb2, gb3 = _bt(edge_b1), _bt(edge_b2), _bt(edge_b3)

    u1h = _bd(upd_W1[:HID])
    u1a = upd_W1[HID:].reshape(NCH, FCH, HID)
    u1lo = jnp.zeros((NCH, FCH, W), _F32).at[:, :, :HID].set(u1a)
    u1hi = jnp.zeros((NCH, FCH, W), _F32).at[:, :, HID:].set(u1a)
    uw2 = _bd(upd_W2)
    uw3 = _bd(upd_W3)
    ub1, ub2, ub3 = _bt(upd_b1), _bt(upd_b2), _bt(upd_b3)

    h2, hp2 = _tc_encoder(x2, ew1, eb1, ew2, eb2, ew3, eb3, w1h)
    for _ in range(MP):
        g2 = _sc_gather(hp2.reshape(N, HID), src2d)
        m2 = _tc_edge(g2, ea2, w1e, gb1, gw2, gb2, gw3, gb3)
        agg = _sc_scatter(m2, dst2d)
        h2, hp2 = _tc_update(h2, agg, u1h, u1lo, u1hi, ub1, uw2, ub2, uw3,
                             ub3, w1h)
    return jnp.concatenate([h2[:, :HID], h2[:, HID:]], axis=0)
